# baseline (speedup 1.0000x reference)
import functools

import jax
import jax.numpy as jnp
from jax import lax
from jax.experimental import pallas as pl
from jax.experimental.pallas import tpu as pltpu
from jax.experimental.pallas import tpu_sc as plsc

_N = 1000
_B = 16384
_EPS = 0.1

_L = 16
_NC = 2
_NS = 16
_NW = _NC * _NS
_BPW = _B // _NW
_R = 32
_NCHUNK = _BPW // _R
_NSLICE = _N // _L


def _sc_body(x_hbm, tgt_hbm, out_hbm, tgt_v, slab0_v, slab1_v, part_v, sem0, sem1):
    wid = lax.axis_index("s") * _NC + lax.axis_index("c")
    base = wid * _BPW
    pltpu.sync_copy(tgt_hbm.at[pl.ds(base, _BPW)], tgt_v)
    lanes = lax.iota(jnp.int32, _L)
    tailmask = lanes >= 8
    zero = jnp.zeros((_L,), jnp.float32)

    pltpu.async_copy(x_hbm.at[pl.ds(base, _R)], slab0_v.at[pl.ds(0, _R)], sem0)
    pltpu.async_copy(
        x_hbm.at[pl.ds(base + _R, _R)], slab1_v.at[pl.ds(0, _R)], sem1
    )
    sems = (sem0, sem1)
    slabs = (slab0_v, slab1_v)

    def process(j, b, carry):
        a0, a1, a2, a3, gsc = carry
        slab = slabs[b]
        pltpu.make_async_copy(
            x_hbm.at[pl.ds(0, _R)], slab.at[pl.ds(0, _R)], sems[b]
        ).wait()

        def row_body(r, rc):
            def cbody(q, cc):
                b0, b1, b2, b3 = cc
                b0 = b0 + slab[r, pl.ds(pl.multiple_of(q * 64, _L), _L)]
                b1 = b1 + slab[r, pl.ds(pl.multiple_of(q * 64 + 16, _L), _L)]
                b2 = b2 + slab[r, pl.ds(pl.multiple_of(q * 64 + 32, _L), _L)]
                b3 = b3 + slab[r, pl.ds(pl.multiple_of(q * 64 + 48, _L), _L)]
                return (b0, b1, b2, b3)

            a0, a1, a2, a3 = lax.fori_loop(0, 60 // 4, cbody, rc)
            a0 = a0 + slab[r, pl.ds(960, _L)]
            a1 = a1 + slab[r, pl.ds(976, _L)]
            vt = slab[r, pl.ds(_N - _L, _L)]
            a3 = a3 + jnp.where(tailmask, vt, zero)
            return (a0, a1, a2, a3)

        a0, a1, a2, a3 = lax.fori_loop(0, _R, row_body, (a0, a1, a2, a3))

        for h in range(_R // _L):
            toff = pl.multiple_of(j * _R + h * _L, _L)
            tv = tgt_v[pl.ds(toff, _L)]
            for k in range(_L):
                t = tv[k]
                c0 = pl.multiple_of((t >> 4) << 4, _L)
                v = slab[h * _L + k, pl.ds(c0, _L)]
                gsc = gsc + jnp.where(lanes == t - c0, v, zero)

        @pl.when(j + 2 < _NCHUNK)
        def _fire_next():
            pltpu.async_copy(
                x_hbm.at[pl.ds(base + (j + 2) * _R, _R)],
                slab.at[pl.ds(0, _R)],
                sems[b],
            )

        return (a0, a1, a2, a3, gsc)

    def outer(p, carry):
        carry = process(2 * p, 0, carry)
        carry = process(2 * p + 1, 1, carry)
        return carry

    init = (zero, zero, zero, zero, zero)
    a0, a1, a2, a3, gsc = lax.fori_loop(0, _NCHUNK // 2, outer, init)
    dsum = (a0 + a1) + (a2 + a3)
    part_v[...] = dsum * (_EPS / _N) + gsc * (1.0 - _EPS)
    pltpu.sync_copy(part_v, out_hbm.at[wid])


_sc_loss = functools.partial(
    pl.kernel,
    out_type=jax.ShapeDtypeStruct((_NW, _L), jnp.float32),
    mesh=plsc.VectorSubcoreMesh(core_axis_name="c", subcore_axis_name="s"),
    scratch_types=[
        pltpu.VMEM((_BPW,), jnp.int32),
        pltpu.VMEM((_R + 1, _N), jnp.float32),
        pltpu.VMEM((_R + 1, _N), jnp.float32),
        pltpu.VMEM((_L,), jnp.float32),
        pltpu.SemaphoreType.DMA,
        pltpu.SemaphoreType.DMA,
    ],
)(_sc_body)


def kernel(inputs, targets):
    targets = targets.astype(jnp.int32)
    partials = _sc_loss(inputs, targets)
    return -jnp.sum(partials) / _B


# speedup vs baseline: 1.0405x; 1.0405x over previous
import functools

import jax
import jax.numpy as jnp
from jax import lax
from jax.experimental import pallas as pl
from jax.experimental.pallas import tpu as pltpu
from jax.experimental.pallas import tpu_sc as plsc

_N = 1000
_B = 16384
_EPS = 0.1

_L = 16
_NC = 2
_NS = 16
_NW = _NC * _NS
_BPW = _B // _NW
_R = 16
_NCHUNK = _BPW // _R
_NSLICE = _N // _L


def _sc_body(x_hbm, tgt_hbm, out_hbm, tgt_v, slab0_v, slab1_v, part_v, sem0, sem1):
    wid = lax.axis_index("s") * _NC + lax.axis_index("c")
    base = wid * _BPW
    pltpu.sync_copy(tgt_hbm.at[pl.ds(base, _BPW)], tgt_v)
    lanes = lax.iota(jnp.int32, _L)
    tailmask = lanes >= 8
    zero = jnp.zeros((_L,), jnp.float32)

    pltpu.async_copy(x_hbm.at[pl.ds(base, _R)], slab0_v.at[pl.ds(0, _R)], sem0)
    pltpu.async_copy(
        x_hbm.at[pl.ds(base + _R, _R)], slab1_v.at[pl.ds(0, _R)], sem1
    )
    sems = (sem0, sem1)
    slabs = (slab0_v, slab1_v)

    def process(j, b, carry):
        a0, a1, a2, a3, gsc = carry
        slab = slabs[b]
        pltpu.make_async_copy(
            x_hbm.at[pl.ds(0, _R)], slab.at[pl.ds(0, _R)], sems[b]
        ).wait()

        def row_body(r, rc):
            accs = list(rc)
            for c in range(_NSLICE):
                v = slab[r, pl.ds(c * _L, _L)]
                accs[c % 4] = accs[c % 4] + v
            vt = slab[r, pl.ds(_N - _L, _L)]
            accs[3] = accs[3] + jnp.where(tailmask, vt, zero)
            return tuple(accs)

        a0, a1, a2, a3 = lax.fori_loop(0, _R, row_body, (a0, a1, a2, a3))

        for h in range(_R // _L):
            toff = pl.multiple_of(j * _R + h * _L, _L)
            tv = tgt_v[pl.ds(toff, _L)]
            for k in range(_L):
                t = tv[k]
                c0 = pl.multiple_of((t >> 4) << 4, _L)
                v = slab[h * _L + k, pl.ds(c0, _L)]
                gsc = gsc + jnp.where(lanes == t - c0, v, zero)

        @pl.when(j + 2 < _NCHUNK)
        def _fire_next():
            pltpu.async_copy(
                x_hbm.at[pl.ds(base + (j + 2) * _R, _R)],
                slab.at[pl.ds(0, _R)],
                sems[b],
            )

        return (a0, a1, a2, a3, gsc)

    def outer(p, carry):
        carry = process(2 * p, 0, carry)
        carry = process(2 * p + 1, 1, carry)
        return carry

    init = (zero, zero, zero, zero, zero)
    a0, a1, a2, a3, gsc = lax.fori_loop(0, _NCHUNK // 2, outer, init)
    dsum = (a0 + a1) + (a2 + a3)
    part_v[...] = dsum * (_EPS / _N) + gsc * (1.0 - _EPS)
    pltpu.sync_copy(part_v, out_hbm.at[wid])


_sc_loss = functools.partial(
    pl.kernel,
    out_type=jax.ShapeDtypeStruct((_NW, _L), jnp.float32),
    mesh=plsc.VectorSubcoreMesh(core_axis_name="c", subcore_axis_name="s"),
    scratch_types=[
        pltpu.VMEM((_BPW,), jnp.int32),
        pltpu.VMEM((_R + 1, _N), jnp.float32),
        pltpu.VMEM((_R + 1, _N), jnp.float32),
        pltpu.VMEM((_L,), jnp.float32),
        pltpu.SemaphoreType.DMA,
        pltpu.SemaphoreType.DMA,
    ],
)(_sc_body)


def kernel(inputs, targets):
    targets = targets.astype(jnp.int32)
    partials = _sc_loss(inputs, targets)
    return -jnp.sum(partials) / _B


# speedup vs baseline: 1.0835x; 1.0413x over previous
import functools

import jax
import jax.numpy as jnp
from jax import lax
from jax.experimental import pallas as pl
from jax.experimental.pallas import tpu as pltpu
from jax.experimental.pallas import tpu_sc as plsc

_N = 1000
_B = 16384
_EPS = 0.1

_L = 16
_NC = 2
_NS = 16
_NW = _NC * _NS
_BPW = _B // _NW
_R = 32
_NCHUNK = _BPW // _R
_NSLICE = _N // _L


def _sc_body(x_hbm, tgt_hbm, out_hbm, tgt_v, slab0_v, slab1_v, part_v, sem0, sem1):
    wid = lax.axis_index("s") * _NC + lax.axis_index("c")
    base = wid * _BPW
    pltpu.sync_copy(tgt_hbm.at[pl.ds(base, _BPW)], tgt_v)
    lanes = lax.iota(jnp.int32, _L)
    tailmask = lanes >= 8
    zero = jnp.zeros((_L,), jnp.float32)

    pltpu.async_copy(x_hbm.at[pl.ds(base, _R)], slab0_v.at[pl.ds(0, _R)], sem0)
    pltpu.async_copy(
        x_hbm.at[pl.ds(base + _R, _R)], slab1_v.at[pl.ds(0, _R)], sem1
    )
    sems = (sem0, sem1)
    slabs = (slab0_v, slab1_v)

    def process(j, b, carry):
        a0, a1, a2, a3, gsc = carry
        slab = slabs[b]
        pltpu.make_async_copy(
            x_hbm.at[pl.ds(0, _R)], slab.at[pl.ds(0, _R)], sems[b]
        ).wait()

        def row_body(r, rc):
            accs = list(rc)
            for c in range(_NSLICE):
                v = slab[r, pl.ds(c * _L, _L)]
                accs[c % 4] = accs[c % 4] + v
            vt = slab[r, pl.ds(_N - _L, _L)]
            accs[3] = accs[3] + jnp.where(tailmask, vt, zero)
            return tuple(accs)

        a0, a1, a2, a3 = lax.fori_loop(0, _R, row_body, (a0, a1, a2, a3))

        for h in range(_R // _L):
            toff = pl.multiple_of(j * _R + h * _L, _L)
            tv = tgt_v[pl.ds(toff, _L)]
            for k in range(_L):
                t = tv[k]
                c0 = pl.multiple_of((t >> 4) << 4, _L)
                v = slab[h * _L + k, pl.ds(c0, _L)]
                gsc = gsc + jnp.where(lanes == t - c0, v, zero)

        @pl.when(j + 2 < _NCHUNK)
        def _fire_next():
            pltpu.async_copy(
                x_hbm.at[pl.ds(base + (j + 2) * _R, _R)],
                slab.at[pl.ds(0, _R)],
                sems[b],
            )

        return (a0, a1, a2, a3, gsc)

    def outer(p, carry):
        carry = process(2 * p, 0, carry)
        carry = process(2 * p + 1, 1, carry)
        return carry

    init = (zero, zero, zero, zero, zero)
    a0, a1, a2, a3, gsc = lax.fori_loop(0, _NCHUNK // 2, outer, init)
    dsum = (a0 + a1) + (a2 + a3)
    part_v[...] = dsum * (_EPS / _N) + gsc * (1.0 - _EPS)
    pltpu.sync_copy(part_v, out_hbm.at[wid])


_sc_loss = functools.partial(
    pl.kernel,
    out_type=jax.ShapeDtypeStruct((_NW, _L), jnp.float32),
    mesh=plsc.VectorSubcoreMesh(core_axis_name="c", subcore_axis_name="s"),
    scratch_types=[
        pltpu.VMEM((_BPW,), jnp.int32),
        pltpu.VMEM((_R + 1, _N), jnp.float32),
        pltpu.VMEM((_R + 1, _N), jnp.float32),
        pltpu.VMEM((_L,), jnp.float32),
        pltpu.SemaphoreType.DMA,
        pltpu.SemaphoreType.DMA,
    ],
)(_sc_body)


def kernel(inputs, targets):
    targets = targets.astype(jnp.int32)
    partials = _sc_loss(inputs, targets)
    return -jnp.sum(partials) / _B
